# baseline (speedup 1.0000x reference)
import numpy as np
import jax
import jax.numpy as jnp
from jax.experimental import pallas as pl
from jax.experimental.pallas import tpu as pltpu

NBINS = 100
R_START = 0.0
R_END = 7.0
BOX = 15.0
CUTOFF = R_END + 0.5
CUT2 = CUTOFF * CUTOFF
HALF = 0.5 * BOX

ROWS = 128
COLT = 128
BPAD = 104
PADV = 1.0e6

_offset64 = np.linspace(R_START, R_END, NBINS)
_width64 = _offset64[1] - _offset64[0]
_sl64 = np.sqrt(0.5 * np.log2(np.e)) / _width64
_offl64 = np.full((BPAD,), 1.0e4, dtype=np.float64)
_offl64[:NBINS] = _sl64 * _offset64
OFF_L = _offl64.astype(np.float32)
SCALE_L = np.float32(_sl64)

_bins64 = np.linspace(R_START, R_END, NBINS + 1)
_vol64 = 4.0 * np.pi / 3.0 * (_bins64[1:] ** 3 - _bins64[:-1] ** 3)
_V64 = 4.0 / 3.0 * np.pi * R_END ** 3
VFAC = (np.float64(_V64) / _vol64).astype(np.float32)
BINS = _bins64.astype(np.float32)


def _hist_kernel(ti_ref, tc_ref, xyz_r_ref, xyz_c_ref, offl_ref, acc_ref,
                 dsc_ref):
    g = pl.program_id(0)
    i = ti_ref[g]
    c = tc_ref[g]

    @pl.when(g == 0)
    def _init():
        acc_ref[...] = jnp.zeros_like(acc_ref)

    def _dist(triangle):
        dsq = jnp.zeros((ROWS, COLT), jnp.float32)
        for dim in range(3):
            xr = xyz_r_ref[:, dim].reshape(ROWS, 1)
            xc = xyz_c_ref[dim, :].reshape(1, COLT)
            t = jnp.abs(xr - xc)
            t = jnp.minimum(t, BOX - t)
            dsq = dsq + t * t
        mask = (dsq < CUT2) & (dsq > 0.0)
        if triangle:
            row_ids = i * ROWS + jax.lax.broadcasted_iota(jnp.int32, (ROWS, COLT), 0)
            col_ids = c * COLT + jax.lax.broadcasted_iota(jnp.int32, (ROWS, COLT), 1)
            mask = mask & (col_ids > row_ids)
        dsc_ref[...] = jnp.where(mask, SCALE_L * jnp.sqrt(dsq), 3.0e4)

    @pl.when(c * COLT >= (i + 1) * ROWS)
    def _dist_above():
        _dist(False)

    @pl.when(c * COLT < (i + 1) * ROWS)
    def _dist_diag():
        _dist(True)

    def _hist():
        def body(r, _):
            drb1 = jnp.broadcast_to(dsc_ref[pl.ds(2 * r, 1), :], (8, COLT))
            drb2 = jnp.broadcast_to(dsc_ref[pl.ds(2 * r + 1, 1), :], (8, COLT))
            for b in range(BPAD // 8):
                s = slice(8 * b, 8 * b + 8)
                o = offl_ref[s, :]
                u1 = drb1 - o
                u2 = drb2 - o
                acc_ref[s, :] += jnp.exp2(-(u1 * u1)) + jnp.exp2(-(u2 * u2))
            return 0

        jax.lax.fori_loop(0, ROWS // 2, body, 0, unroll=8)

    _hist()


def _fin_kernel(acc_ref, vfac_ref, count_ref, rdf_ref):
    counts = jnp.sum(acc_ref[0:NBINS, :], axis=1).reshape(1, NBINS)
    norm = jnp.sum(counts)
    cn = counts / norm
    count_ref[...] = cn
    rdf_ref[...] = cn * vfac_ref[...]


def _rdf_call(xyz_pad, xyz_pad_t, offl, vfac):
    npad = xyz_pad.shape[0]
    tiles = [(i, c)
             for i in range(npad // ROWS)
             for c in range(npad // COLT)
             if (c + 1) * COLT > i * ROWS]
    ti = jnp.asarray([t[0] for t in tiles], jnp.int32)
    tc = jnp.asarray([t[1] for t in tiles], jnp.int32)
    acc = pl.pallas_call(
        _hist_kernel,
        grid_spec=pltpu.PrefetchScalarGridSpec(
            num_scalar_prefetch=2,
            grid=(len(tiles),),
            in_specs=[
                pl.BlockSpec((ROWS, 3), lambda g, ti, tc: (ti[g], 0)),
                pl.BlockSpec((3, COLT), lambda g, ti, tc: (0, tc[g])),
                pl.BlockSpec((BPAD, COLT), lambda g, ti, tc: (0, 0)),
            ],
            out_specs=pl.BlockSpec((BPAD, COLT), lambda g, ti, tc: (0, 0)),
            scratch_shapes=[
                pltpu.VMEM((ROWS, COLT), jnp.float32),
            ],
        ),
        out_shape=jax.ShapeDtypeStruct((BPAD, COLT), jnp.float32),
    )(ti, tc, xyz_pad, xyz_pad_t, offl)
    count, rdf = pl.pallas_call(
        _fin_kernel,
        out_shape=[
            jax.ShapeDtypeStruct((1, NBINS), jnp.float32),
            jax.ShapeDtypeStruct((1, NBINS), jnp.float32),
        ],
    )(acc, vfac)
    return count, rdf


def kernel(xyz):
    n = xyz.shape[0]
    npad = ((n + COLT - 1) // COLT) * COLT
    xyz_pad = jnp.full((npad, 3), PADV, jnp.float32).at[:n, :].set(xyz)
    xyz_pad_t = xyz_pad.T
    offl = jnp.broadcast_to(jnp.asarray(OFF_L)[:, None], (BPAD, COLT))
    vfac = jnp.asarray(VFAC).reshape(1, NBINS)
    count, rdf = _rdf_call(xyz_pad, xyz_pad_t, offl, vfac)
    bins = jnp.asarray(BINS)
    return count.reshape(NBINS), bins, rdf.reshape(NBINS)


# speedup vs baseline: 1.0367x; 1.0367x over previous
import numpy as np
import jax
import jax.numpy as jnp
from jax.experimental import pallas as pl
from jax.experimental.pallas import tpu as pltpu

NBINS = 100
R_START = 0.0
R_END = 7.0
BOX = 15.0
CUTOFF = R_END + 0.5
CUT2 = CUTOFF * CUTOFF
HALF = 0.5 * BOX

ROWS = 128
COLT = 256
BPAD = 104
PADV = 1.0e6

_offset64 = np.linspace(R_START, R_END, NBINS)
_width64 = _offset64[1] - _offset64[0]
_sl64 = np.sqrt(0.5 * np.log2(np.e)) / _width64
_offl64 = np.full((BPAD,), 1.0e4, dtype=np.float64)
_offl64[:NBINS] = _sl64 * _offset64
OFF_L = _offl64.astype(np.float32)
SCALE_L = np.float32(_sl64)

_bins64 = np.linspace(R_START, R_END, NBINS + 1)
_vol64 = 4.0 * np.pi / 3.0 * (_bins64[1:] ** 3 - _bins64[:-1] ** 3)
_V64 = 4.0 / 3.0 * np.pi * R_END ** 3
VFAC = (np.float64(_V64) / _vol64).astype(np.float32)
BINS = _bins64.astype(np.float32)


def _hist_kernel(ti_ref, tc_ref, xyz_r_ref, xyz_c_ref, offl_ref, acc_ref,
                 dsc_ref):
    g = pl.program_id(0)
    i = ti_ref[g]
    c = tc_ref[g]

    @pl.when(g == 0)
    def _init():
        acc_ref[...] = jnp.zeros_like(acc_ref)

    def _dist(triangle):
        dsq = jnp.zeros((ROWS, COLT), jnp.float32)
        for dim in range(3):
            xr = xyz_r_ref[:, dim].reshape(ROWS, 1)
            xc = xyz_c_ref[dim, :].reshape(1, COLT)
            t = jnp.abs(xr - xc)
            t = jnp.minimum(t, BOX - t)
            dsq = dsq + t * t
        mask = (dsq < CUT2) & (dsq > 0.0)
        if triangle:
            row_ids = i * ROWS + jax.lax.broadcasted_iota(jnp.int32, (ROWS, COLT), 0)
            col_ids = c * COLT + jax.lax.broadcasted_iota(jnp.int32, (ROWS, COLT), 1)
            mask = mask & (col_ids > row_ids)
        dsc_ref[...] = jnp.where(mask, SCALE_L * jnp.sqrt(dsq), 3.0e4)

    @pl.when(c * COLT >= (i + 1) * ROWS)
    def _dist_above():
        _dist(False)

    @pl.when(c * COLT < (i + 1) * ROWS)
    def _dist_diag():
        _dist(True)

    def _hist():
        def body(r, _):
            drb1 = jnp.broadcast_to(dsc_ref[pl.ds(2 * r, 1), :], (8, COLT))
            drb2 = jnp.broadcast_to(dsc_ref[pl.ds(2 * r + 1, 1), :], (8, COLT))
            for b in range(BPAD // 8):
                s = slice(8 * b, 8 * b + 8)
                o = offl_ref[s, :]
                u1 = drb1 - o
                u2 = drb2 - o
                acc_ref[s, :] += jnp.exp2(-(u1 * u1)) + jnp.exp2(-(u2 * u2))
            return 0

        jax.lax.fori_loop(0, ROWS // 2, body, 0, unroll=8)

    _hist()


def _fin_kernel(acc_ref, vfac_ref, count_ref, rdf_ref):
    counts = jnp.sum(acc_ref[0:NBINS, :], axis=1).reshape(1, NBINS)
    norm = jnp.sum(counts)
    cn = counts / norm
    count_ref[...] = cn
    rdf_ref[...] = cn * vfac_ref[...]


def _rdf_call(xyz_pad, xyz_pad_t, offl, vfac):
    npad = xyz_pad.shape[0]
    tiles = [(i, c)
             for i in range(npad // ROWS)
             for c in range(npad // COLT)
             if (c + 1) * COLT > i * ROWS]
    ti = jnp.asarray([t[0] for t in tiles], jnp.int32)
    tc = jnp.asarray([t[1] for t in tiles], jnp.int32)
    acc = pl.pallas_call(
        _hist_kernel,
        grid_spec=pltpu.PrefetchScalarGridSpec(
            num_scalar_prefetch=2,
            grid=(len(tiles),),
            in_specs=[
                pl.BlockSpec((ROWS, 3), lambda g, ti, tc: (ti[g], 0)),
                pl.BlockSpec((3, COLT), lambda g, ti, tc: (0, tc[g])),
                pl.BlockSpec((BPAD, COLT), lambda g, ti, tc: (0, 0)),
            ],
            out_specs=pl.BlockSpec((BPAD, COLT), lambda g, ti, tc: (0, 0)),
            scratch_shapes=[
                pltpu.VMEM((ROWS, COLT), jnp.float32),
            ],
        ),
        out_shape=jax.ShapeDtypeStruct((BPAD, COLT), jnp.float32),
    )(ti, tc, xyz_pad, xyz_pad_t, offl)
    count, rdf = pl.pallas_call(
        _fin_kernel,
        out_shape=[
            jax.ShapeDtypeStruct((1, NBINS), jnp.float32),
            jax.ShapeDtypeStruct((1, NBINS), jnp.float32),
        ],
    )(acc, vfac)
    return count, rdf


def kernel(xyz):
    n = xyz.shape[0]
    npad = ((n + COLT - 1) // COLT) * COLT
    xyz_pad = jnp.full((npad, 3), PADV, jnp.float32).at[:n, :].set(xyz)
    xyz_pad_t = xyz_pad.T
    offl = jnp.broadcast_to(jnp.asarray(OFF_L)[:, None], (BPAD, COLT))
    vfac = jnp.asarray(VFAC).reshape(1, NBINS)
    count, rdf = _rdf_call(xyz_pad, xyz_pad_t, offl, vfac)
    bins = jnp.asarray(BINS)
    return count.reshape(NBINS), bins, rdf.reshape(NBINS)


# speedup vs baseline: 1.0591x; 1.0216x over previous
import numpy as np
import jax
import jax.numpy as jnp
from jax.experimental import pallas as pl
from jax.experimental.pallas import tpu as pltpu

NBINS = 100
R_START = 0.0
R_END = 7.0
BOX = 15.0
CUTOFF = R_END + 0.5
CUT2 = CUTOFF * CUTOFF
HALF = 0.5 * BOX

ROWS = 256
COLT = 256
BPAD = 104
PADV = 1.0e6

_offset64 = np.linspace(R_START, R_END, NBINS)
_width64 = _offset64[1] - _offset64[0]
_sl64 = np.sqrt(0.5 * np.log2(np.e)) / _width64
_offl64 = np.full((BPAD,), 1.0e4, dtype=np.float64)
_offl64[:NBINS] = _sl64 * _offset64
OFF_L = _offl64.astype(np.float32)
SCALE_L = np.float32(_sl64)

_bins64 = np.linspace(R_START, R_END, NBINS + 1)
_vol64 = 4.0 * np.pi / 3.0 * (_bins64[1:] ** 3 - _bins64[:-1] ** 3)
_V64 = 4.0 / 3.0 * np.pi * R_END ** 3
VFAC = (np.float64(_V64) / _vol64).astype(np.float32)
BINS = _bins64.astype(np.float32)


def _hist_kernel(ti_ref, tc_ref, xyz_r_ref, xyz_c_ref, offl_ref, acc_ref,
                 dsc_ref):
    g = pl.program_id(0)
    i = ti_ref[g]
    c = tc_ref[g]

    @pl.when(g == 0)
    def _init():
        acc_ref[...] = jnp.zeros_like(acc_ref)

    def _dist(triangle):
        dsq = jnp.zeros((ROWS, COLT), jnp.float32)
        for dim in range(3):
            xr = xyz_r_ref[:, dim].reshape(ROWS, 1)
            xc = xyz_c_ref[dim, :].reshape(1, COLT)
            t = jnp.abs(xr - xc)
            t = jnp.minimum(t, BOX - t)
            dsq = dsq + t * t
        mask = (dsq < CUT2) & (dsq > 0.0)
        if triangle:
            row_ids = i * ROWS + jax.lax.broadcasted_iota(jnp.int32, (ROWS, COLT), 0)
            col_ids = c * COLT + jax.lax.broadcasted_iota(jnp.int32, (ROWS, COLT), 1)
            mask = mask & (col_ids > row_ids)
        dsc_ref[...] = jnp.where(mask, SCALE_L * jnp.sqrt(dsq), 3.0e4)

    @pl.when(c * COLT >= (i + 1) * ROWS)
    def _dist_above():
        _dist(False)

    @pl.when(c * COLT < (i + 1) * ROWS)
    def _dist_diag():
        _dist(True)

    def _hist():
        def body(r, _):
            drb1 = jnp.broadcast_to(dsc_ref[pl.ds(2 * r, 1), :], (8, COLT))
            drb2 = jnp.broadcast_to(dsc_ref[pl.ds(2 * r + 1, 1), :], (8, COLT))
            for b in range(BPAD // 8):
                s = slice(8 * b, 8 * b + 8)
                o = offl_ref[s, :]
                u1 = drb1 - o
                u2 = drb2 - o
                acc_ref[s, :] += jnp.exp2(-(u1 * u1)) + jnp.exp2(-(u2 * u2))
            return 0

        jax.lax.fori_loop(0, ROWS // 2, body, 0, unroll=8)

    _hist()


def _fin_kernel(acc_ref, vfac_ref, count_ref, rdf_ref):
    counts = jnp.sum(acc_ref[0:NBINS, :], axis=1).reshape(1, NBINS)
    norm = jnp.sum(counts)
    cn = counts / norm
    count_ref[...] = cn
    rdf_ref[...] = cn * vfac_ref[...]


def _rdf_call(xyz_pad, xyz_pad_t, offl, vfac):
    npad = xyz_pad.shape[0]
    tiles = [(i, c)
             for i in range(npad // ROWS)
             for c in range(npad // COLT)
             if (c + 1) * COLT > i * ROWS]
    ti = jnp.asarray([t[0] for t in tiles], jnp.int32)
    tc = jnp.asarray([t[1] for t in tiles], jnp.int32)
    acc = pl.pallas_call(
        _hist_kernel,
        grid_spec=pltpu.PrefetchScalarGridSpec(
            num_scalar_prefetch=2,
            grid=(len(tiles),),
            in_specs=[
                pl.BlockSpec((ROWS, 3), lambda g, ti, tc: (ti[g], 0)),
                pl.BlockSpec((3, COLT), lambda g, ti, tc: (0, tc[g])),
                pl.BlockSpec((BPAD, COLT), lambda g, ti, tc: (0, 0)),
            ],
            out_specs=pl.BlockSpec((BPAD, COLT), lambda g, ti, tc: (0, 0)),
            scratch_shapes=[
                pltpu.VMEM((ROWS, COLT), jnp.float32),
            ],
        ),
        out_shape=jax.ShapeDtypeStruct((BPAD, COLT), jnp.float32),
    )(ti, tc, xyz_pad, xyz_pad_t, offl)
    count, rdf = pl.pallas_call(
        _fin_kernel,
        out_shape=[
            jax.ShapeDtypeStruct((1, NBINS), jnp.float32),
            jax.ShapeDtypeStruct((1, NBINS), jnp.float32),
        ],
    )(acc, vfac)
    return count, rdf


def kernel(xyz):
    n = xyz.shape[0]
    npad = ((n + COLT - 1) // COLT) * COLT
    xyz_pad = jnp.full((npad, 3), PADV, jnp.float32).at[:n, :].set(xyz)
    xyz_pad_t = xyz_pad.T
    offl = jnp.broadcast_to(jnp.asarray(OFF_L)[:, None], (BPAD, COLT))
    vfac = jnp.asarray(VFAC).reshape(1, NBINS)
    count, rdf = _rdf_call(xyz_pad, xyz_pad_t, offl, vfac)
    bins = jnp.asarray(BINS)
    return count.reshape(NBINS), bins, rdf.reshape(NBINS)
